# baseline (speedup 1.0000x reference)
import functools

import jax
import jax.numpy as jnp
from jax import lax
from jax.experimental import pallas as pl
from jax.experimental.pallas import tpu as pltpu
from jax.experimental.pallas import tpu_sc as plsc

BATCH = 16384
DIM = 1024
B_SC = 9216
B_TC = BATCH - B_SC

_info = plsc.get_sparse_core_info()
_NC, _NS, _L = _info.num_cores, _info.num_subcores, _info.num_lanes
NW = _NC * _NS
ROWS_PER_W = B_SC // NW
R = 8
NBUF = 4
NCHUNK = ROWS_PER_W // R


def _x_body(x_hbm, pa_hbm, pb_hbm, xo_hbm, pa_v, pb_v, *rest):
    xins = rest[0:NBUF]
    xouts = rest[NBUF:2 * NBUF]
    sxis = rest[2 * NBUF:3 * NBUF]
    sxos = rest[3 * NBUF:4 * NBUF]

    wid = lax.axis_index("s") * _NC + lax.axis_index("c")
    row_base = wid * ROWS_PER_W
    nct = DIM // 128

    pltpu.sync_copy(pa_hbm, pa_v)
    pltpu.sync_copy(pb_hbm, pb_v)

    def issue_in(ci, k):
        row = row_base + ci * R
        for ct in range(nct):
            pltpu.async_copy(
                x_hbm.at[pl.ds(row, 8), pl.ds(ct * 128, 128)],
                xins[k].at[pl.ds(ct * 8, 8)], sxis[k])

    def wait_in(k):
        for ct in range(nct):
            pltpu.make_async_copy(
                x_hbm.at[pl.ds(0, 8), pl.ds(0, 128)],
                xins[k].at[pl.ds(0, 8)], sxis[k]).wait()

    def issue_out(ci, k):
        row = row_base + ci * R
        for ct in range(nct):
            pltpu.async_copy(
                xouts[k].at[pl.ds(ct * 8, 8)],
                xo_hbm.at[pl.ds(row, 8), pl.ds(ct * 128, 128)], sxos[k])

    def wait_out(k):
        for ct in range(nct):
            pltpu.make_async_copy(
                xouts[k].at[pl.ds(0, 8)],
                xo_hbm.at[pl.ds(0, 8), pl.ds(0, 128)], sxos[k]).wait()

    def compute(k):
        xin, xout = xins[k], xouts[k]

        @plsc.parallel_loop(0, DIM // _L, unroll=2)
        def _(g):
            pav = pa_v[pl.ds(g * _L, _L)]
            pbv = pb_v[pl.ds(g * _L, _L)]
            for r in range(R):
                av = pav + r
                v = plsc.load_gather(xin, [av, pbv])
                a_out = (g // 8) * 8 + r
                xout[a_out, pl.ds((g % 8) * _L, _L)] = v

    for ci in range(NBUF):
        issue_in(ci, ci)
    for ci in range(NBUF):
        wait_in(ci)
        compute(ci)
        issue_out(ci, ci)
        issue_in(ci + NBUF, ci)

    def outer(it, carry):
        cb = NBUF + it * NBUF
        for k in range(NBUF):
            ci = cb + k
            wait_in(k)
            wait_out(k)
            compute(k)
            issue_out(ci, k)

            @pl.when(ci + NBUF < NCHUNK)
            def _():
                issue_in(ci + NBUF, k)
        return carry
    lax.fori_loop(0, (NCHUNK - NBUF) // NBUF, outer, 0, unroll=1)

    for k in range(NBUF):
        wait_out(k)


_mesh = plsc.VectorSubcoreMesh(core_axis_name="c", subcore_axis_name="s")

_x_call = functools.partial(
    pl.kernel,
    out_type=jax.ShapeDtypeStruct((BATCH, DIM), jnp.float32),
    mesh=_mesh,
    compiler_params=pltpu.CompilerParams(needs_layout_passes=False),
    scratch_types=(
        [pltpu.VMEM((DIM,), jnp.int32)] * 2
        + [pltpu.VMEM((R * 8, 128), jnp.float32)] * (2 * NBUF)
        + [pltpu.SemaphoreType.DMA] * (2 * NBUF)
    ),
)


MROWS = 1024


def _mask_body(perm_ref, m_ref, out_ref, p_scratch):
    @pl.when(pl.program_id(0) == 0)
    def _():
        iota = lax.broadcasted_iota(jnp.int32, (DIM, DIM), 0)
        p_scratch[...] = (iota == perm_ref[0][None, :]).astype(jnp.bfloat16)

    m = m_ref[...].astype(jnp.bfloat16)
    acc = jnp.dot(m, p_scratch[...], preferred_element_type=jnp.float32)
    out_ref[...] = acc > 0.5


_mask_call = pl.pallas_call(
    _mask_body,
    grid=(BATCH // MROWS,),
    in_specs=[
        pl.BlockSpec((1, DIM), lambda i: (0, 0)),
        pl.BlockSpec((MROWS, DIM), lambda i: (i, 0)),
    ],
    out_specs=pl.BlockSpec((MROWS, DIM), lambda i: (i, 0)),
    out_shape=jax.ShapeDtypeStruct((BATCH, DIM), jnp.bool_),
    scratch_shapes=[pltpu.VMEM((DIM, DIM), jnp.bfloat16)],
)


XROWS = 1024


def _xtc_body(perm_ref, x_ref, thru_ref, out_ref, p_scratch):
    del thru_ref
    @pl.when(pl.program_id(0) == 0)
    def _():
        iota = lax.broadcasted_iota(jnp.int32, (DIM, DIM), 0)
        p_scratch[...] = (iota == perm_ref[0][None, :]).astype(jnp.bfloat16)

    p = p_scratch[...]
    xb = x_ref[...]
    hi = xb.astype(jnp.bfloat16)
    r1 = xb - hi.astype(jnp.float32)
    lo = r1.astype(jnp.bfloat16)
    rest = (r1 - lo.astype(jnp.float32)).astype(jnp.bfloat16)
    acc = jnp.dot(hi, p, preferred_element_type=jnp.float32)
    acc += jnp.dot(lo, p, preferred_element_type=jnp.float32)
    acc += jnp.dot(rest, p, preferred_element_type=jnp.float32)
    out_ref[...] = acc


_xtc_call = pl.pallas_call(
    _xtc_body,
    grid=(B_TC // XROWS,),
    in_specs=[
        pl.BlockSpec((1, DIM), lambda i: (0, 0)),
        pl.BlockSpec((XROWS, DIM), lambda i: (B_SC // XROWS + i, 0)),
        pl.BlockSpec(memory_space=pltpu.MemorySpace.HBM),
    ],
    out_specs=pl.BlockSpec((XROWS, DIM), lambda i: (B_SC // XROWS + i, 0)),
    out_shape=jax.ShapeDtypeStruct((BATCH, DIM), jnp.float32),
    input_output_aliases={2: 0},
    scratch_shapes=[pltpu.VMEM((DIM, DIM), jnp.bfloat16)],
)


def kernel(x, observed_mask, perm, inv_perm):
    del inv_perm
    pa8 = ((perm >> 7) << 3).astype(jnp.int32)
    pb = (perm & 127).astype(jnp.int32)
    x_sc = _x_call(_x_body)(x, pa8, pb)
    perm2d = perm.reshape(1, DIM)
    m_out = _mask_call(perm2d, observed_mask)
    x_out = _xtc_call(perm2d, x, x_sc)
    return (x_out, m_out)


# speedup vs baseline: 1.2216x; 1.2216x over previous
import functools

import jax
import jax.numpy as jnp
from jax import lax
from jax.experimental import pallas as pl
from jax.experimental.pallas import tpu as pltpu
from jax.experimental.pallas import tpu_sc as plsc

BATCH = 16384
DIM = 1024

_info = plsc.get_sparse_core_info()
_NC, _NS, _L = _info.num_cores, _info.num_subcores, _info.num_lanes
NW = _NC * _NS
ROWS_PER_W = BATCH // NW
R = 8
NBUF = 4
NCHUNK = ROWS_PER_W // R


def _x_body(x_hbm, pa_hbm, pb_hbm, xo_hbm, pa_v, pb_v, *rest):
    xins = rest[0:NBUF]
    xouts = rest[NBUF:2 * NBUF]
    sxis = rest[2 * NBUF:3 * NBUF]
    sxos = rest[3 * NBUF:4 * NBUF]

    wid = lax.axis_index("s") * _NC + lax.axis_index("c")
    row_base = wid * ROWS_PER_W
    nct = DIM // 128

    pltpu.sync_copy(pa_hbm, pa_v)
    pltpu.sync_copy(pb_hbm, pb_v)

    def issue_in(ci, k):
        row = row_base + ci * R
        for ct in range(nct):
            pltpu.async_copy(
                x_hbm.at[pl.ds(row, 8), pl.ds(ct * 128, 128)],
                xins[k].at[pl.ds(ct * 8, 8)], sxis[k])

    def wait_in(k):
        for ct in range(nct):
            pltpu.make_async_copy(
                x_hbm.at[pl.ds(0, 8), pl.ds(0, 128)],
                xins[k].at[pl.ds(0, 8)], sxis[k]).wait()

    def issue_out(ci, k):
        row = row_base + ci * R
        for ct in range(nct):
            pltpu.async_copy(
                xouts[k].at[pl.ds(ct * 8, 8)],
                xo_hbm.at[pl.ds(row, 8), pl.ds(ct * 128, 128)], sxos[k])

    def wait_out(k):
        for ct in range(nct):
            pltpu.make_async_copy(
                xouts[k].at[pl.ds(0, 8)],
                xo_hbm.at[pl.ds(0, 8), pl.ds(0, 128)], sxos[k]).wait()

    def compute(k):
        xin, xout = xins[k], xouts[k]

        @plsc.parallel_loop(0, DIM // _L, unroll=2)
        def _(g):
            pav = pa_v[pl.ds(g * _L, _L)]
            pbv = pb_v[pl.ds(g * _L, _L)]
            for r in range(R):
                av = pav + r
                v = plsc.load_gather(xin, [av, pbv])
                a_out = (g // 8) * 8 + r
                xout[a_out, pl.ds((g % 8) * _L, _L)] = v

    for ci in range(NBUF):
        issue_in(ci, ci)
    for ci in range(NBUF):
        wait_in(ci)
        compute(ci)
        issue_out(ci, ci)
        issue_in(ci + NBUF, ci)

    def outer(it, carry):
        cb = NBUF + it * NBUF
        for k in range(NBUF):
            ci = cb + k
            wait_in(k)
            wait_out(k)
            compute(k)
            issue_out(ci, k)

            @pl.when(ci + NBUF < NCHUNK)
            def _():
                issue_in(ci + NBUF, k)
        return carry
    lax.fori_loop(0, (NCHUNK - NBUF) // NBUF, outer, 0, unroll=1)

    for k in range(NBUF):
        wait_out(k)


_mesh = plsc.VectorSubcoreMesh(core_axis_name="c", subcore_axis_name="s")

_x_call = functools.partial(
    pl.kernel,
    out_type=jax.ShapeDtypeStruct((BATCH, DIM), jnp.float32),
    mesh=_mesh,
    compiler_params=pltpu.CompilerParams(needs_layout_passes=False),
    scratch_types=(
        [pltpu.VMEM((DIM,), jnp.int32)] * 2
        + [pltpu.VMEM((R * 8, 128), jnp.float32)] * (2 * NBUF)
        + [pltpu.SemaphoreType.DMA] * (2 * NBUF)
    ),
)


MROWS = 1024


def _mask_body(perm_ref, m_ref, out_ref, p_scratch):
    @pl.when(pl.program_id(0) == 0)
    def _():
        iota = lax.broadcasted_iota(jnp.int32, (DIM, DIM), 0)
        p_scratch[...] = (iota == perm_ref[0][None, :]).astype(jnp.bfloat16)

    m = m_ref[...].astype(jnp.bfloat16)
    acc = jnp.dot(m, p_scratch[...], preferred_element_type=jnp.float32)
    out_ref[...] = acc > 0.5


_mask_call = pl.pallas_call(
    _mask_body,
    grid=(BATCH // MROWS,),
    in_specs=[
        pl.BlockSpec((1, DIM), lambda i: (0, 0)),
        pl.BlockSpec((MROWS, DIM), lambda i: (i, 0)),
    ],
    out_specs=pl.BlockSpec((MROWS, DIM), lambda i: (i, 0)),
    out_shape=jax.ShapeDtypeStruct((BATCH, DIM), jnp.bool_),
    scratch_shapes=[pltpu.VMEM((DIM, DIM), jnp.bfloat16)],
)


def kernel(x, observed_mask, perm, inv_perm):
    del inv_perm
    pa8 = ((perm >> 7) << 3).astype(jnp.int32)
    pb = (perm & 127).astype(jnp.int32)
    x_out = _x_call(_x_body)(x, pa8, pb)
    m_out = _mask_call(perm.reshape(1, DIM), observed_mask)
    return (x_out, m_out)
